# Initial kernel scaffold; baseline (speedup 1.0000x reference)
import jax
import jax.numpy as jnp
from jax.experimental import pallas as pl


def kernel(x, edge_index, W1, b1, W2, b2):
    raise NotImplementedError("write your pallas kernel here")


# speedup vs baseline: 10.5108x; 10.5108x over previous
import functools

import jax
import jax.numpy as jnp
from jax import lax
from jax.experimental import pallas as pl
from jax.experimental.pallas import tpu as pltpu
from jax.experimental.pallas import tpu_sc as plsc

NC = 2
NS = 16
L = 16
NW = NC * NS
CHUNK = 128
DC = 128


def _sc_agg(NP, D, e_per_w, n_chunks):
    mesh = plsc.VectorSubcoreMesh(core_axis_name="c", subcore_axis_name="s")
    rpt = NP // NS

    @functools.partial(
        pl.kernel,
        mesh=mesh,
        out_type=jax.ShapeDtypeStruct((NC, NP, D), jnp.float32),
        scratch_types=[
            pltpu.VMEM_SHARED((NP, D), jnp.float32),
            pltpu.VMEM((CHUNK,), jnp.int32),
            pltpu.VMEM((CHUNK,), jnp.int32),
            pltpu.VMEM((CHUNK, D), jnp.float32),
            pltpu.VMEM((CHUNK, D), jnp.float32),
            pltpu.SemaphoreType.DMA,
        ],
    )
    def k(xs_hbm, src_hbm, dst_hbm, out_hbm, acc, idx_s, idx_d, rows, zbuf, sem):
        c = lax.axis_index("c")
        s = lax.axis_index("s")
        wid = c * NS + s

        def fill_zero(i, _):
            for j in range(D // L):
                zbuf[i, pl.ds(j * L, L)] = jnp.zeros((L,), jnp.float32)
            return 0

        lax.fori_loop(0, CHUNK, fill_zero, 0)
        for j in range(rpt // CHUNK):
            pltpu.sync_copy(zbuf, acc.at[pl.ds(s * rpt + j * CHUNK, CHUNK)])
        plsc.subcore_barrier()

        base = wid * e_per_w

        def body(t, _):
            off = base + t * CHUNK
            pltpu.sync_copy(src_hbm.at[pl.ds(off, CHUNK)], idx_s)
            pltpu.sync_copy(dst_hbm.at[pl.ds(off, CHUNK)], idx_d)
            pltpu.async_copy(xs_hbm.at[idx_s], rows, sem).wait()
            pltpu.sync_copy(rows, acc.at[idx_d], add=True)
            return 0

        lax.fori_loop(0, n_chunks, body, 0)
        plsc.subcore_barrier()
        pltpu.sync_copy(acc.at[pl.ds(s * rpt, rpt)],
                        out_hbm.at[c, pl.ds(s * rpt, rpt)])

    return k


def _sc_deg(NP, e_per_w, n_chunks):
    mesh = plsc.VectorSubcoreMesh(core_axis_name="c", subcore_axis_name="s")
    rpt = NP // NS

    @functools.partial(
        pl.kernel,
        mesh=mesh,
        out_type=jax.ShapeDtypeStruct((NC, NP, DC), jnp.float32),
        scratch_types=[
            pltpu.VMEM_SHARED((NP, DC), jnp.float32),
            pltpu.VMEM((CHUNK,), jnp.int32),
            pltpu.VMEM((CHUNK, DC), jnp.float32),
            pltpu.VMEM((CHUNK, DC), jnp.float32),
            pltpu.SemaphoreType.DMA,
        ],
    )
    def k(dst_hbm, out_hbm, acc, idx_d, ones, zbuf, sem):
        c = lax.axis_index("c")
        s = lax.axis_index("s")
        wid = c * NS + s

        def fill(i, _):
            for j in range(DC // L):
                zbuf[i, pl.ds(j * L, L)] = jnp.zeros((L,), jnp.float32)
                ones[i, pl.ds(j * L, L)] = jnp.ones((L,), jnp.float32)
            return 0

        lax.fori_loop(0, CHUNK, fill, 0)
        for j in range(rpt // CHUNK):
            pltpu.sync_copy(zbuf, acc.at[pl.ds(s * rpt + j * CHUNK, CHUNK)])
        plsc.subcore_barrier()

        base = wid * e_per_w

        def body(t, _):
            off = base + t * CHUNK
            pltpu.sync_copy(dst_hbm.at[pl.ds(off, CHUNK)], idx_d)
            pltpu.sync_copy(ones, acc.at[idx_d], add=True)
            return 0

        lax.fori_loop(0, n_chunks, body, 0)
        plsc.subcore_barrier()
        pltpu.sync_copy(acc.at[pl.ds(s * rpt, rpt)],
                        out_hbm.at[c, pl.ds(s * rpt, rpt)])

    return k


def _tc_scale(x_blk, d0_blk, d1_blk, xs_out, dinv_out):
    deg = 1.0 + d0_blk[:, :1] + d1_blk[:, :1]
    dinv = lax.rsqrt(deg)
    dinv_out[...] = jnp.broadcast_to(dinv, xs_out.shape)
    xs_out[...] = x_blk[...] * dinv


def _tc_mid(p0, p1, xs, dinv, W1, b1, W2, ts_out):
    ax = dinv[...] * (p0[...] + p1[...] + xs[...])
    h1 = jnp.maximum(
        jnp.dot(ax, W1[...], preferred_element_type=jnp.float32) + b1[...], 0.0)
    t = jnp.dot(h1, W2[...], preferred_element_type=jnp.float32)
    ts_out[...] = t * dinv[...]


def _tc_final(q0, q1, ts, dinv, b2, out):
    out[...] = dinv[...] * (q0[...] + q1[...] + ts[...]) + b2[...]


def kernel(x, edge_index, W1, b1, W2, b2):
    N, d_in = x.shape
    E = edge_index.shape[1]
    d_hid = W1.shape[1]
    d_out = W2.shape[1]

    NP = ((N + 1 + NS * CHUNK - 1) // (NS * CHUNK)) * (NS * CHUNK)
    Epad = ((E + NW * CHUNK - 1) // (NW * CHUNK)) * (NW * CHUNK)
    e_per_w = Epad // NW
    n_chunks = e_per_w // CHUNK

    src = jnp.concatenate(
        [edge_index[0], jnp.full((Epad - E,), N, jnp.int32)])
    dst = jnp.concatenate(
        [edge_index[1], jnp.full((Epad - E,), N, jnp.int32)])
    x_pad = jnp.pad(x, ((0, NP - N), (0, 0)))

    degp = _sc_deg(NP, e_per_w, n_chunks)(dst)

    R = 512
    grid = (NP // R,)
    blk = lambda d: pl.BlockSpec((R, d), lambda i: (i, 0))
    full = lambda shape: pl.BlockSpec(shape, lambda i: tuple(0 for _ in shape))

    xs, dinv = pl.pallas_call(
        _tc_scale,
        grid=grid,
        in_specs=[blk(d_in), pl.BlockSpec((R, DC), lambda i: (i, 0)),
                  pl.BlockSpec((R, DC), lambda i: (i, 0))],
        out_specs=[blk(d_in), blk(d_in)],
        out_shape=[jax.ShapeDtypeStruct((NP, d_in), jnp.float32),
                   jax.ShapeDtypeStruct((NP, d_in), jnp.float32)],
    )(x_pad, degp[0], degp[1])

    p = _sc_agg(NP, d_in, e_per_w, n_chunks)(xs, src, dst)

    ts = pl.pallas_call(
        _tc_mid,
        grid=grid,
        in_specs=[blk(d_in), blk(d_in), blk(d_in), blk(d_in),
                  full((d_in, d_hid)), full((1, d_hid)), full((d_hid, d_out))],
        out_specs=blk(d_out),
        out_shape=jax.ShapeDtypeStruct((NP, d_out), jnp.float32),
    )(p[0], p[1], xs, dinv, W1, b1.reshape(1, d_hid), W2)

    q = _sc_agg(NP, d_out, e_per_w, n_chunks)(ts, src, dst)

    out = pl.pallas_call(
        _tc_final,
        grid=grid,
        in_specs=[blk(d_out), blk(d_out), blk(d_out), blk(d_out),
                  full((1, d_out))],
        out_specs=blk(d_out),
        out_shape=jax.ShapeDtypeStruct((NP, d_out), jnp.float32),
    )(q[0], q[1], ts, dinv, b2.reshape(1, d_out))

    return out[:N]


# speedup vs baseline: 28.2264x; 2.6855x over previous
import functools

import jax
import jax.numpy as jnp
from jax import lax
from jax.experimental import pallas as pl
from jax.experimental.pallas import tpu as pltpu
from jax.experimental.pallas import tpu_sc as plsc

NC = 2
NS = 16
L = 16
NW = NC * NS
CHUNK = 128
DC = 128
NB = 4
ZR = 16


def _sc_agg(NP, D, e_per_w, n_chunks):
    mesh = plsc.VectorSubcoreMesh(core_axis_name="c", subcore_axis_name="s")
    rpt = NP // NS
    T = n_chunks

    @functools.partial(
        pl.kernel,
        mesh=mesh,
        out_type=jax.ShapeDtypeStruct((NC, NP, D), jnp.float32),
        scratch_types=[
            pltpu.VMEM_SHARED((NP, D), jnp.float32),
            pltpu.VMEM((2, CHUNK), jnp.int32),
            pltpu.VMEM((n_chunks, CHUNK), jnp.int32),
            pltpu.VMEM((2, CHUNK, D), jnp.float32),
            pltpu.VMEM((ZR, D), jnp.float32),
        ] + [pltpu.SemaphoreType.DMA] * 6,
    )
    def k(xs_hbm, src_hbm, dst3_hbm, out_hbm, acc, sidx, didx, rows, zbuf,
          *sems):
        isem, gsem, ssem = sems[0:2], sems[2:4], sems[4:6]
        c = lax.axis_index("c")
        s = lax.axis_index("s")
        wid = c * NS + s
        base = wid * e_per_w

        pltpu.sync_copy(dst3_hbm.at[wid], didx)

        def fill_zero(i, _):
            for j in range(D // L):
                zbuf[i, pl.ds(j * L, L)] = jnp.zeros((L,), jnp.float32)
            return 0

        lax.fori_loop(0, ZR, fill_zero, 0)

        def zero_blk(j, _):
            pltpu.sync_copy(zbuf, acc.at[pl.ds(s * rpt + j * ZR, ZR)])
            return 0

        lax.fori_loop(0, rpt // ZR, zero_blk, 0)
        plsc.subcore_barrier()

        def idx_desc(t, b):
            return pltpu.make_async_copy(
                src_hbm.at[pl.ds(base + t * CHUNK, CHUNK)], sidx.at[b],
                isem[b])

        def g_desc(b):
            return pltpu.make_async_copy(
                xs_hbm.at[sidx.at[b]], rows.at[b], gsem[b])

        def fire_scatter(t, b):
            pltpu.async_copy(rows.at[b], acc.at[didx.at[t]], ssem[b],
                             add=True)

        def wait_scatter(t, b):
            pltpu.make_async_copy(rows.at[b], acc.at[didx.at[t]],
                                  ssem[b]).wait()

        idx_desc(0, 0).start()
        idx_desc(1, 1).start()
        idx_desc(0, 0).wait()
        g_desc(0).start()
        idx_desc(1, 1).wait()
        g_desc(1).start()
        g_desc(0).wait()
        idx_desc(2, 0).start()
        fire_scatter(0, 0)

        def body(g, _):
            t0 = 2 * g
            wait_scatter(t0 - 2, 0)
            idx_desc(t0, 0).wait()
            g_desc(0).start()
            g_desc(1).wait()
            idx_desc(jnp.minimum(t0 + 1, T - 1), 1).start()
            fire_scatter(t0 - 1, 1)
            wait_scatter(t0 - 1, 1)
            idx_desc(t0 + 1, 1).wait()
            g_desc(1).start()
            g_desc(0).wait()
            idx_desc(jnp.minimum(t0 + 2, T - 1), 0).start()
            fire_scatter(t0, 0)
            return 0

        lax.fori_loop(1, T // 2, body, 0)
        g_desc(1).wait()
        fire_scatter(T - 1, 1)
        idx_desc(T - 1, 0).wait()
        wait_scatter(T - 2, 0)
        wait_scatter(T - 1, 1)

        plsc.subcore_barrier()
        pltpu.sync_copy(acc.at[pl.ds(s * rpt, rpt)],
                        out_hbm.at[c, pl.ds(s * rpt, rpt)])

    return k


def _sc_deg(NP, e_per_w, n_chunks):
    mesh = plsc.VectorSubcoreMesh(core_axis_name="c", subcore_axis_name="s")
    rpt = NP // NS
    n_super = n_chunks // NB

    @functools.partial(
        pl.kernel,
        mesh=mesh,
        out_type=jax.ShapeDtypeStruct((NC, NP, DC), jnp.float32),
        scratch_types=[
            pltpu.VMEM_SHARED((NP, DC), jnp.float32),
            pltpu.VMEM((n_chunks, CHUNK), jnp.int32),
            pltpu.VMEM((CHUNK, DC), jnp.float32),
            pltpu.VMEM((ZR, DC), jnp.float32),
        ] + [pltpu.SemaphoreType.DMA] * NB,
    )
    def k(dst3_hbm, out_hbm, acc, didx, ones, zbuf, *ssem):
        c = lax.axis_index("c")
        s = lax.axis_index("s")
        wid = c * NS + s

        pltpu.sync_copy(dst3_hbm.at[wid], didx)

        def fill_ones(i, _):
            for j in range(DC // L):
                ones[i, pl.ds(j * L, L)] = jnp.ones((L,), jnp.float32)
            return 0

        def fill_zero(i, _):
            for j in range(DC // L):
                zbuf[i, pl.ds(j * L, L)] = jnp.zeros((L,), jnp.float32)
            return 0

        lax.fori_loop(0, CHUNK, fill_ones, 0)
        lax.fori_loop(0, ZR, fill_zero, 0)

        def zero_blk(j, _):
            pltpu.sync_copy(zbuf, acc.at[pl.ds(s * rpt + j * ZR, ZR)])
            return 0

        lax.fori_loop(0, rpt // ZR, zero_blk, 0)
        plsc.subcore_barrier()

        def scatter(t, b):
            pltpu.async_copy(ones, acc.at[didx.at[t]], ssem[b], add=True)

        def wait_s(t, b):
            pltpu.make_async_copy(ones, acc.at[didx.at[t]], ssem[b]).wait()

        for b in range(NB):
            scatter(b, b)

        def body(g, _):
            for b in range(NB):
                t = g * NB + b
                wait_s(t, b)
                scatter(t + NB, b)
            return 0

        lax.fori_loop(0, n_super - 1, body, 0)
        for b in range(NB):
            t = (n_super - 1) * NB + b
            wait_s(t, b)

        plsc.subcore_barrier()
        pltpu.sync_copy(acc.at[pl.ds(s * rpt, rpt)],
                        out_hbm.at[c, pl.ds(s * rpt, rpt)])

    return k


def _tc_scale(x_blk, d0_blk, d1_blk, xs_out, dinv_out):
    deg = 1.0 + d0_blk[:, :1] + d1_blk[:, :1]
    dinv = lax.rsqrt(deg)
    dinv_out[...] = jnp.broadcast_to(dinv, xs_out.shape)
    xs_out[...] = x_blk[...] * dinv


def _tc_mid(p0, p1, xs, dinv, W1, b1, W2, ts_out):
    ax = dinv[...] * (p0[...] + p1[...] + xs[...])
    h1 = jnp.maximum(
        jnp.dot(ax, W1[...], preferred_element_type=jnp.float32) + b1[...], 0.0)
    t = jnp.dot(h1, W2[...], preferred_element_type=jnp.float32)
    ts_out[...] = t * dinv[...]


def _tc_final(q0, q1, ts, dinv, b2, out):
    out[...] = dinv[...] * (q0[...] + q1[...] + ts[...]) + b2[...]


def kernel(x, edge_index, W1, b1, W2, b2):
    N, d_in = x.shape
    E = edge_index.shape[1]
    d_hid = W1.shape[1]
    d_out = W2.shape[1]

    NP = ((N + 1 + NS * CHUNK - 1) // (NS * CHUNK)) * (NS * CHUNK)
    Epad = ((E + NW * CHUNK * NB - 1) // (NW * CHUNK * NB)) * (NW * CHUNK * NB)
    e_per_w = Epad // NW
    n_chunks = e_per_w // CHUNK

    pad_idx = (N + jnp.arange(Epad - E, dtype=jnp.int32) % (NP - N)).astype(
        jnp.int32)
    src = jnp.concatenate([edge_index[0], pad_idx])
    dst = jnp.concatenate([edge_index[1], pad_idx])
    dst3 = dst.reshape(NW, n_chunks, CHUNK)
    x_pad = jnp.pad(x, ((0, NP - N), (0, 0)))

    degp = _sc_deg(NP, e_per_w, n_chunks)(dst3)

    R = 512
    grid = (NP // R,)
    blk = lambda d: pl.BlockSpec((R, d), lambda i: (i, 0))
    full = lambda shape: pl.BlockSpec(shape, lambda i: tuple(0 for _ in shape))

    xs, dinv = pl.pallas_call(
        _tc_scale,
        grid=grid,
        in_specs=[blk(d_in), pl.BlockSpec((R, DC), lambda i: (i, 0)),
                  pl.BlockSpec((R, DC), lambda i: (i, 0))],
        out_specs=[blk(d_in), blk(d_in)],
        out_shape=[jax.ShapeDtypeStruct((NP, d_in), jnp.float32),
                   jax.ShapeDtypeStruct((NP, d_in), jnp.float32)],
    )(x_pad, degp[0], degp[1])

    p = _sc_agg(NP, d_in, e_per_w, n_chunks)(xs, src, dst3)

    ts = pl.pallas_call(
        _tc_mid,
        grid=grid,
        in_specs=[blk(d_in), blk(d_in), blk(d_in), blk(d_in),
                  full((d_in, d_hid)), full((1, d_hid)), full((d_hid, d_out))],
        out_specs=blk(d_out),
        out_shape=jax.ShapeDtypeStruct((NP, d_out), jnp.float32),
    )(p[0], p[1], xs, dinv, W1, b1.reshape(1, d_hid), W2)

    q = _sc_agg(NP, d_out, e_per_w, n_chunks)(ts, src, dst3)

    out = pl.pallas_call(
        _tc_final,
        grid=grid,
        in_specs=[blk(d_out), blk(d_out), blk(d_out), blk(d_out),
                  full((1, d_out))],
        out_specs=blk(d_out),
        out_shape=jax.ShapeDtypeStruct((NP, d_out), jnp.float32),
    )(q[0], q[1], ts, dinv, b2.reshape(1, d_out))

    return out[:N]


# speedup vs baseline: 29.5971x; 1.0486x over previous
import functools

import jax
import jax.numpy as jnp
from jax import lax
from jax.experimental import pallas as pl
from jax.experimental.pallas import tpu as pltpu
from jax.experimental.pallas import tpu_sc as plsc

NC = 2
NS = 16
L = 16
NW = NC * NS
CHUNK = 120
DC = 128
NB = 4
ZR = 16


def _sc_agg(NP, D, e_per_w, n_chunks):
    mesh = plsc.VectorSubcoreMesh(core_axis_name="c", subcore_axis_name="s")
    rpt = NP // NS
    T = n_chunks
    ZC = rpt // CHUNK
    ZT = rpt - ZC * CHUNK

    @functools.partial(
        pl.kernel,
        mesh=mesh,
        out_type=jax.ShapeDtypeStruct((NC, NP, D), jnp.float32),
        scratch_types=[
            pltpu.VMEM_SHARED((NP, D), jnp.float32),
            pltpu.VMEM((6, 2, CHUNK), jnp.int32),
            pltpu.VMEM((3, CHUNK, D), jnp.float32),
        ] + [pltpu.SemaphoreType.DMA] * 12,
    )
    def k(xs_hbm, idx3_hbm, out_hbm, acc, idx, rows, *sems):
        isem, gsem, ssem = sems[0:6], sems[6:9], sems[9:12]
        c = lax.axis_index("c")
        s = lax.axis_index("s")
        wid = c * NS + s

        def fill_zero(i, _):
            for j in range(D // L):
                rows[0, i, pl.ds(j * L, L)] = jnp.zeros((L,), jnp.float32)
            return 0

        lax.fori_loop(0, CHUNK, fill_zero, 0)
        for j in range(ZC):
            pltpu.sync_copy(rows.at[0],
                            acc.at[pl.ds(s * rpt + j * CHUNK, CHUNK)])
        if ZT:
            pltpu.sync_copy(rows.at[0, pl.ds(0, ZT)],
                            acc.at[pl.ds(s * rpt + ZC * CHUNK, ZT)])
        plsc.subcore_barrier()

        def idx_desc(t, i6):
            return pltpu.make_async_copy(idx3_hbm.at[wid, t], idx.at[i6],
                                         isem[i6])

        def g_desc(t, b):
            return pltpu.make_async_copy(xs_hbm.at[idx.at[t % 6, 0]],
                                         rows.at[b], gsem[b])

        def s_fire(t, b):
            pltpu.async_copy(rows.at[b], acc.at[idx.at[t % 6, 1]], ssem[b],
                             add=True)

        def s_wait(t, b):
            pltpu.make_async_copy(rows.at[b], acc.at[idx.at[t % 6, 1]],
                                  ssem[b]).wait()

        def visit(t):
            b = t % 3
            s_wait(t - 3, b)
            idx_desc(t, t % 6).wait()
            g_desc(t, b).start()
            g_desc(t - 1, (t - 1) % 3).wait()
            idx_desc(min(t + 2, T - 1), (t + 2) % 6).start()
            s_fire(t - 1, (t - 1) % 3)

        for t in range(3):
            idx_desc(t, t).start()
        idx_desc(0, 0).wait()
        g_desc(0, 0).start()
        for t in (1, 2):
            idx_desc(t, t).wait()
            g_desc(t, t).start()
            g_desc(t - 1, t - 1).wait()
            idx_desc(t + 2, (t + 2) % 6).start()
            s_fire(t - 1, t - 1)
        for t in (3, 4, 5):
            visit(t)

        def body(g, _):
            t0 = 6 * g
            for r in range(6):
                b = r % 3
                i6p = (t0 + r - 3) % 6
                pltpu.make_async_copy(rows.at[b], acc.at[idx.at[i6p, 1]],
                                      ssem[b]).wait()
                idx_desc(t0 + r, r).wait()
                g_desc(t0 + r, b).start()
                bp = (r - 1) % 3
                pltpu.make_async_copy(
                    xs_hbm.at[idx.at[(r - 1) % 6, 0]], rows.at[bp],
                    gsem[bp]).wait()
                idx_desc(jnp.minimum(t0 + r + 2, T - 1), (r + 2) % 6).start()
                pltpu.async_copy(rows.at[bp], acc.at[idx.at[(r - 1) % 6, 1]],
                                 ssem[bp], add=True)
            return 0

        lax.fori_loop(1, T // 6, body, 0)

        bl = (T - 1) % 3
        g_desc(T - 1, bl).wait()
        s_fire(T - 1, bl)
        for t in (T - 3, T - 2, T - 1):
            s_wait(t, t % 3)
        for u in (T % 6, (T + 1) % 6):
            pltpu.make_async_copy(idx3_hbm.at[wid, T - 1], idx.at[u],
                                  isem[u]).wait()

        plsc.subcore_barrier()
        pltpu.sync_copy(acc.at[pl.ds(s * rpt, rpt)],
                        out_hbm.at[c, pl.ds(s * rpt, rpt)])

    return k


def _sc_deg(NP, e_per_w, n_chunks):
    mesh = plsc.VectorSubcoreMesh(core_axis_name="c", subcore_axis_name="s")
    rpt = NP // NS
    n_super = n_chunks // NB

    @functools.partial(
        pl.kernel,
        mesh=mesh,
        out_type=jax.ShapeDtypeStruct((NC, NP, DC), jnp.float32),
        scratch_types=[
            pltpu.VMEM_SHARED((NP, DC), jnp.float32),
            pltpu.VMEM((n_chunks, CHUNK), jnp.int32),
            pltpu.VMEM((CHUNK, DC), jnp.float32),
            pltpu.VMEM((ZR, DC), jnp.float32),
        ] + [pltpu.SemaphoreType.DMA] * NB,
    )
    def k(dst3_hbm, out_hbm, acc, didx, ones, zbuf, *ssem):
        c = lax.axis_index("c")
        s = lax.axis_index("s")
        wid = c * NS + s

        pltpu.sync_copy(dst3_hbm.at[wid], didx)

        def fill_ones(i, _):
            for j in range(DC // L):
                ones[i, pl.ds(j * L, L)] = jnp.ones((L,), jnp.float32)
            return 0

        def fill_zero(i, _):
            for j in range(DC // L):
                zbuf[i, pl.ds(j * L, L)] = jnp.zeros((L,), jnp.float32)
            return 0

        lax.fori_loop(0, CHUNK, fill_ones, 0)
        lax.fori_loop(0, ZR, fill_zero, 0)

        def zero_blk(j, _):
            pltpu.sync_copy(zbuf, acc.at[pl.ds(s * rpt + j * ZR, ZR)])
            return 0

        lax.fori_loop(0, rpt // ZR, zero_blk, 0)
        if rpt % ZR:
            pltpu.sync_copy(
                zbuf.at[pl.ds(0, rpt % ZR)],
                acc.at[pl.ds(s * rpt + (rpt // ZR) * ZR, rpt % ZR)])
        plsc.subcore_barrier()

        def scatter(t, b):
            pltpu.async_copy(ones, acc.at[didx.at[t]], ssem[b], add=True)

        def wait_s(t, b):
            pltpu.make_async_copy(ones, acc.at[didx.at[t]], ssem[b]).wait()

        for b in range(NB):
            scatter(b, b)

        def body(g, _):
            for b in range(NB):
                t = g * NB + b
                wait_s(t, b)
                scatter(t + NB, b)
            return 0

        lax.fori_loop(0, n_super - 1, body, 0)
        for b in range(NB):
            t = (n_super - 1) * NB + b
            wait_s(t, b)

        plsc.subcore_barrier()
        pltpu.sync_copy(acc.at[pl.ds(s * rpt, rpt)],
                        out_hbm.at[c, pl.ds(s * rpt, rpt)])

    return k


def _tc_scale(x_blk, d0_blk, d1_blk, xs_out, dinv_out):
    deg = 1.0 + d0_blk[:, :1] + d1_blk[:, :1]
    dinv = lax.rsqrt(deg)
    dinv_out[...] = jnp.broadcast_to(dinv, xs_out.shape)
    xs_out[...] = x_blk[...] * dinv


def _tc_mid(p0, p1, xs, dinv, W1, b1, W2, ts_out):
    ax = dinv[...] * (p0[...] + p1[...] + xs[...])
    h1 = jnp.maximum(
        jnp.dot(ax, W1[...], preferred_element_type=jnp.float32) + b1[...], 0.0)
    t = jnp.dot(h1, W2[...], preferred_element_type=jnp.float32)
    ts_out[...] = t * dinv[...]


def _tc_final(q0, q1, ts, dinv, b2, out):
    out[...] = dinv[...] * (q0[...] + q1[...] + ts[...]) + b2[...]


def kernel(x, edge_index, W1, b1, W2, b2):
    N, d_in = x.shape
    E = edge_index.shape[1]
    d_hid = W1.shape[1]
    d_out = W2.shape[1]

    NP = ((N + 1 + NS * 8 - 1) // (NS * 8)) * (NS * 8)
    Epad = ((E + NW * CHUNK * 6 - 1) // (NW * CHUNK * 6)) * (NW * CHUNK * 6)
    e_per_w = Epad // NW
    n_chunks = e_per_w // CHUNK
    Eh = ((E + NW * CHUNK * NB - 1) // (NW * CHUNK * NB)) * (NW * CHUNK * NB)
    eh_per_w = Eh // NW
    nh_chunks = eh_per_w // CHUNK

    pad_idx = (N + jnp.arange(Epad - E, dtype=jnp.int32) % (NP - N)).astype(
        jnp.int32)
    src = jnp.concatenate([edge_index[0], pad_idx])
    dst = jnp.concatenate([edge_index[1], pad_idx])
    idx3 = jnp.stack([src.reshape(NW, n_chunks, CHUNK),
                      dst.reshape(NW, n_chunks, CHUNK)], axis=2)
    dst3h = jnp.concatenate(
        [edge_index[1], pad_idx[:Eh - E]]).reshape(NW, nh_chunks, CHUNK)
    x_pad = jnp.pad(x, ((0, NP - N), (0, 0)))

    degp = _sc_deg(NP, eh_per_w, nh_chunks)(dst3h)

    R = 512
    grid = (pl.cdiv(NP, R),)
    blk = lambda d: pl.BlockSpec((R, d), lambda i: (i, 0))
    full = lambda shape: pl.BlockSpec(shape, lambda i: tuple(0 for _ in shape))

    xs, dinv = pl.pallas_call(
        _tc_scale,
        grid=grid,
        in_specs=[blk(d_in), pl.BlockSpec((R, DC), lambda i: (i, 0)),
                  pl.BlockSpec((R, DC), lambda i: (i, 0))],
        out_specs=[blk(d_in), blk(d_in)],
        out_shape=[jax.ShapeDtypeStruct((NP, d_in), jnp.float32),
                   jax.ShapeDtypeStruct((NP, d_in), jnp.float32)],
    )(x_pad, degp[0], degp[1])

    p = _sc_agg(NP, d_in, e_per_w, n_chunks)(xs, idx3)

    ts = pl.pallas_call(
        _tc_mid,
        grid=grid,
        in_specs=[blk(d_in), blk(d_in), blk(d_in), blk(d_in),
                  full((d_in, d_hid)), full((1, d_hid)), full((d_hid, d_out))],
        out_specs=blk(d_out),
        out_shape=jax.ShapeDtypeStruct((NP, d_out), jnp.float32),
    )(p[0], p[1], xs, dinv, W1, b1.reshape(1, d_hid), W2)

    q = _sc_agg(NP, d_out, e_per_w, n_chunks)(ts, idx3)

    out = pl.pallas_call(
        _tc_final,
        grid=grid,
        in_specs=[blk(d_out), blk(d_out), blk(d_out), blk(d_out),
                  full((1, d_out))],
        out_specs=blk(d_out),
        out_shape=jax.ShapeDtypeStruct((NP, d_out), jnp.float32),
    )(q[0], q[1], ts, dinv, b2.reshape(1, d_out))

    return out[:N]


# speedup vs baseline: 31.3440x; 1.0590x over previous
import functools

import jax
import jax.numpy as jnp
from jax import lax
from jax.experimental import pallas as pl
from jax.experimental.pallas import tpu as pltpu
from jax.experimental.pallas import tpu_sc as plsc

NC = 2
NS = 16
L = 16
NW = NC * NS
CHUNK = 120
DC = 128
NB = 4
ZR = 16


def _sc_agg(NP, D, e_per_w, n_chunks):
    mesh = plsc.VectorSubcoreMesh(core_axis_name="c", subcore_axis_name="s")
    rpt = NP // NS
    T = n_chunks
    ZC = rpt // CHUNK
    ZT = rpt - ZC * CHUNK

    @functools.partial(
        pl.kernel,
        mesh=mesh,
        out_type=jax.ShapeDtypeStruct((NC, NP, D), jnp.float32),
        scratch_types=[
            pltpu.VMEM_SHARED((NP, D), jnp.float32),
            pltpu.VMEM((6, 2, CHUNK), jnp.int32),
            pltpu.VMEM((3, CHUNK, D), jnp.float32),
        ] + [pltpu.SemaphoreType.DMA] * 12,
    )
    def k(xs_hbm, idx3_hbm, out_hbm, acc, idx, rows, *sems):
        isem, gsem, ssem = sems[0:6], sems[6:9], sems[9:12]
        c = lax.axis_index("c")
        s = lax.axis_index("s")
        wid = c * NS + s

        def fill_zero(i, _):
            for j in range(D // L):
                rows[0, i, pl.ds(j * L, L)] = jnp.zeros((L,), jnp.float32)
            return 0

        lax.fori_loop(0, CHUNK, fill_zero, 0)
        for j in range(ZC):
            pltpu.sync_copy(rows.at[0],
                            acc.at[pl.ds(s * rpt + j * CHUNK, CHUNK)])
        if ZT:
            pltpu.sync_copy(rows.at[0, pl.ds(0, ZT)],
                            acc.at[pl.ds(s * rpt + ZC * CHUNK, ZT)])
        plsc.subcore_barrier()

        def idx_desc(t, i6):
            return pltpu.make_async_copy(idx3_hbm.at[wid, t], idx.at[i6],
                                         isem[i6])

        def g_desc(t, b):
            return pltpu.make_async_copy(xs_hbm.at[idx.at[t % 6, 0]],
                                         rows.at[b], gsem[b])

        def s_fire(t, b):
            pltpu.async_copy(rows.at[b], acc.at[idx.at[t % 6, 1]], ssem[b],
                             add=True)

        def s_wait(t, b):
            pltpu.make_async_copy(rows.at[b], acc.at[idx.at[t % 6, 1]],
                                  ssem[b]).wait()

        def visit(t):
            b = t % 3
            s_wait(t - 3, b)
            idx_desc(t, t % 6).wait()
            g_desc(t, b).start()
            g_desc(t - 1, (t - 1) % 3).wait()
            idx_desc(min(t + 2, T - 1), (t + 2) % 6).start()
            s_fire(t - 1, (t - 1) % 3)

        for t in range(3):
            idx_desc(t, t).start()
        idx_desc(0, 0).wait()
        g_desc(0, 0).start()
        for t in (1, 2):
            idx_desc(t, t).wait()
            g_desc(t, t).start()
            g_desc(t - 1, t - 1).wait()
            idx_desc(t + 2, (t + 2) % 6).start()
            s_fire(t - 1, t - 1)
        for t in (3, 4, 5):
            visit(t)

        def body(g, _):
            t0 = 6 * g
            for r in range(6):
                b = r % 3
                i6p = (t0 + r - 3) % 6
                pltpu.make_async_copy(rows.at[b], acc.at[idx.at[i6p, 1]],
                                      ssem[b]).wait()
                idx_desc(t0 + r, r).wait()
                g_desc(t0 + r, b).start()
                bp = (r - 1) % 3
                pltpu.make_async_copy(
                    xs_hbm.at[idx.at[(r - 1) % 6, 0]], rows.at[bp],
                    gsem[bp]).wait()
                idx_desc(jnp.minimum(t0 + r + 2, T - 1), (r + 2) % 6).start()
                pltpu.async_copy(rows.at[bp], acc.at[idx.at[(r - 1) % 6, 1]],
                                 ssem[bp], add=True)
            return 0

        lax.fori_loop(1, T // 6, body, 0)

        bl = (T - 1) % 3
        g_desc(T - 1, bl).wait()
        s_fire(T - 1, bl)
        for t in (T - 3, T - 2, T - 1):
            s_wait(t, t % 3)
        for u in (T % 6, (T + 1) % 6):
            pltpu.make_async_copy(idx3_hbm.at[wid, T - 1], idx.at[u],
                                  isem[u]).wait()

        plsc.subcore_barrier()
        pltpu.sync_copy(acc.at[pl.ds(s * rpt, rpt)],
                        out_hbm.at[c, pl.ds(s * rpt, rpt)])

    return k


def _sc_deg(NP, e_per_w, n_chunks):
    mesh = plsc.VectorSubcoreMesh(core_axis_name="c", subcore_axis_name="s")
    rpt = NP // NS
    n_super = n_chunks // NB

    @functools.partial(
        pl.kernel,
        mesh=mesh,
        out_type=jax.ShapeDtypeStruct((NC, NP, DC), jnp.float32),
        scratch_types=[
            pltpu.VMEM_SHARED((NP, DC), jnp.float32),
            pltpu.VMEM((n_chunks, CHUNK), jnp.int32),
            pltpu.VMEM((CHUNK, DC), jnp.float32),
            pltpu.VMEM((ZR, DC), jnp.float32),
        ] + [pltpu.SemaphoreType.DMA] * NB,
    )
    def k(dst3_hbm, out_hbm, acc, didx, ones, zbuf, *ssem):
        c = lax.axis_index("c")
        s = lax.axis_index("s")
        wid = c * NS + s

        pltpu.sync_copy(dst3_hbm.at[wid], didx)

        def fill_ones(i, _):
            for j in range(DC // L):
                ones[i, pl.ds(j * L, L)] = jnp.ones((L,), jnp.float32)
            return 0

        def fill_zero(i, _):
            for j in range(DC // L):
                zbuf[i, pl.ds(j * L, L)] = jnp.zeros((L,), jnp.float32)
            return 0

        lax.fori_loop(0, CHUNK, fill_ones, 0)
        lax.fori_loop(0, ZR, fill_zero, 0)

        def zero_blk(j, _):
            pltpu.sync_copy(zbuf, acc.at[pl.ds(s * rpt + j * ZR, ZR)])
            return 0

        lax.fori_loop(0, rpt // ZR, zero_blk, 0)
        if rpt % ZR:
            pltpu.sync_copy(
                zbuf.at[pl.ds(0, rpt % ZR)],
                acc.at[pl.ds(s * rpt + (rpt // ZR) * ZR, rpt % ZR)])
        plsc.subcore_barrier()

        def scatter(t, b):
            pltpu.async_copy(ones, acc.at[didx.at[t]], ssem[b], add=True)

        def wait_s(t, b):
            pltpu.make_async_copy(ones, acc.at[didx.at[t]], ssem[b]).wait()

        for b in range(NB):
            scatter(b, b)

        def body(g, _):
            for b in range(NB):
                t = g * NB + b
                wait_s(t, b)
                scatter(t + NB, b)
            return 0

        lax.fori_loop(0, n_super - 1, body, 0)
        for b in range(NB):
            t = (n_super - 1) * NB + b
            wait_s(t, b)

        plsc.subcore_barrier()
        pltpu.sync_copy(acc.at[pl.ds(s * rpt, rpt)],
                        out_hbm.at[c, pl.ds(s * rpt, rpt)])

    return k


def _tc_scale(x_blk, dp_blk, xs_out, dinv_out):
    deg = 1.0 + dp_blk[0, :, :1] + dp_blk[1, :, :1]
    dinv = lax.rsqrt(deg)
    dinv_out[...] = jnp.broadcast_to(dinv, xs_out.shape)
    xs_out[...] = x_blk[...] * dinv


def _tc_mid(p_blk, xs, dinv, W1, b1, W2, ts_out):
    ax = dinv[...] * (p_blk[0] + p_blk[1] + xs[...])
    h1 = jnp.maximum(
        jnp.dot(ax, W1[...], preferred_element_type=jnp.float32) + b1[...], 0.0)
    t = jnp.dot(h1, W2[...], preferred_element_type=jnp.float32)
    ts_out[...] = t * dinv[...]


def _tc_final(q_blk, ts, dinv, b2, out):
    out[...] = dinv[...] * (q_blk[0] + q_blk[1] + ts[...]) + b2[...]


def kernel(x, edge_index, W1, b1, W2, b2):
    N, d_in = x.shape
    E = edge_index.shape[1]
    d_hid = W1.shape[1]
    d_out = W2.shape[1]

    NP = ((N + 1 + NS * 8 - 1) // (NS * 8)) * (NS * 8)
    Epad = ((E + NW * CHUNK * 6 - 1) // (NW * CHUNK * 6)) * (NW * CHUNK * 6)
    e_per_w = Epad // NW
    n_chunks = e_per_w // CHUNK
    Eh = ((E + NW * CHUNK * NB - 1) // (NW * CHUNK * NB)) * (NW * CHUNK * NB)
    eh_per_w = Eh // NW
    nh_chunks = eh_per_w // CHUNK

    pad_idx = (N + jnp.arange(Epad - E, dtype=jnp.int32) % (NP - N)).astype(
        jnp.int32)
    src = jnp.concatenate([edge_index[0], pad_idx])
    dst = jnp.concatenate([edge_index[1], pad_idx])
    idx3 = jnp.stack([src.reshape(NW, n_chunks, CHUNK),
                      dst.reshape(NW, n_chunks, CHUNK)], axis=2)
    dst3h = jnp.concatenate(
        [edge_index[1], pad_idx[:Eh - E]]).reshape(NW, nh_chunks, CHUNK)
    x_pad = jnp.pad(x, ((0, NP - N), (0, 0)))

    degp = _sc_deg(NP, eh_per_w, nh_chunks)(dst3h)

    R = 512
    grid = (pl.cdiv(NP, R),)
    blk = lambda d: pl.BlockSpec((R, d), lambda i: (i, 0))
    blk2 = lambda d: pl.BlockSpec((2, R, d), lambda i: (0, i, 0))
    full = lambda shape: pl.BlockSpec(shape, lambda i: tuple(0 for _ in shape))

    xs, dinv = pl.pallas_call(
        _tc_scale,
        grid=grid,
        in_specs=[blk(d_in), blk2(DC)],
        out_specs=[blk(d_in), blk(d_in)],
        out_shape=[jax.ShapeDtypeStruct((NP, d_in), jnp.float32),
                   jax.ShapeDtypeStruct((NP, d_in), jnp.float32)],
    )(x_pad, degp)

    p = _sc_agg(NP, d_in, e_per_w, n_chunks)(xs, idx3)

    ts = pl.pallas_call(
        _tc_mid,
        grid=grid,
        in_specs=[blk2(d_in), blk(d_in), blk(d_in),
                  full((d_in, d_hid)), full((1, d_hid)), full((d_hid, d_out))],
        out_specs=blk(d_out),
        out_shape=jax.ShapeDtypeStruct((NP, d_out), jnp.float32),
    )(p, xs, dinv, W1, b1.reshape(1, d_hid), W2)

    q = _sc_agg(NP, d_out, e_per_w, n_chunks)(ts, idx3)

    out = pl.pallas_call(
        _tc_final,
        grid=grid,
        in_specs=[blk2(d_out), blk(d_out), blk(d_out),
                  full((1, d_out))],
        out_specs=blk(d_out),
        out_shape=jax.ShapeDtypeStruct((NP, d_out), jnp.float32),
    )(q, ts, dinv, b2.reshape(1, d_out))

    return out[:N]


# speedup vs baseline: 31.3772x; 1.0011x over previous
import functools

import jax
import jax.numpy as jnp
from jax import lax
from jax.experimental import pallas as pl
from jax.experimental.pallas import tpu as pltpu
from jax.experimental.pallas import tpu_sc as plsc

NC = 2
NS = 16
L = 16
NW = NC * NS
CHUNK = 120
DC = 128
NB = 4


def _sc_agg(NP, D, e_per_w, n_chunks):
    mesh = plsc.VectorSubcoreMesh(core_axis_name="c", subcore_axis_name="s")
    rpt = NP // NS
    T = n_chunks
    ZC = rpt // CHUNK
    ZT = rpt - ZC * CHUNK

    @functools.partial(
        pl.kernel,
        mesh=mesh,
        out_type=jax.ShapeDtypeStruct((NC, NP, D), jnp.float32),
        scratch_types=[
            pltpu.VMEM_SHARED((NP, D), jnp.float32),
            pltpu.VMEM((6, 2, CHUNK), jnp.int32),
            pltpu.VMEM((3, CHUNK, D), jnp.float32),
        ] + [pltpu.SemaphoreType.DMA] * 12,
    )
    def k(xs_hbm, idx3_hbm, out_hbm, acc, idx, rows, *sems):
        isem, gsem, ssem = sems[0:6], sems[6:9], sems[9:12]
        c = lax.axis_index("c")
        s = lax.axis_index("s")
        wid = c * NS + s

        def fill_zero(i, _):
            for j in range(D // L):
                rows[0, i, pl.ds(j * L, L)] = jnp.zeros((L,), jnp.float32)
            return 0

        lax.fori_loop(0, CHUNK, fill_zero, 0)
        for j in range(ZC):
            pltpu.sync_copy(rows.at[0],
                            acc.at[pl.ds(s * rpt + j * CHUNK, CHUNK)])
        if ZT:
            pltpu.sync_copy(rows.at[0, pl.ds(0, ZT)],
                            acc.at[pl.ds(s * rpt + ZC * CHUNK, ZT)])
        plsc.subcore_barrier()

        def idx_desc(t, i6):
            return pltpu.make_async_copy(idx3_hbm.at[wid, t], idx.at[i6],
                                         isem[i6])

        def g_desc(t, b):
            return pltpu.make_async_copy(xs_hbm.at[idx.at[t % 6, 0]],
                                         rows.at[b], gsem[b])

        def s_fire(t, b):
            pltpu.async_copy(rows.at[b], acc.at[idx.at[t % 6, 1]], ssem[b],
                             add=True)

        def s_wait(t, b):
            pltpu.make_async_copy(rows.at[b], acc.at[idx.at[t % 6, 1]],
                                  ssem[b]).wait()

        def visit(t):
            b = t % 3
            s_wait(t - 3, b)
            idx_desc(t, t % 6).wait()
            g_desc(t, b).start()
            g_desc(t - 1, (t - 1) % 3).wait()
            idx_desc(min(t + 2, T - 1), (t + 2) % 6).start()
            s_fire(t - 1, (t - 1) % 3)

        for t in range(3):
            idx_desc(t, t).start()
        idx_desc(0, 0).wait()
        g_desc(0, 0).start()
        for t in (1, 2):
            idx_desc(t, t).wait()
            g_desc(t, t).start()
            idx_desc(t + 2, (t + 2) % 6).start()
            g_desc(t - 1, t - 1).wait()
            s_fire(t - 1, t - 1)
        for t in (3, 4, 5):
            visit(t)

        def body(g, _):
            t0 = 6 * g
            for r in range(6):
                b = r % 3
                i6p = (t0 + r - 3) % 6
                pltpu.make_async_copy(rows.at[b], acc.at[idx.at[i6p, 1]],
                                      ssem[b]).wait()
                idx_desc(t0 + r, r).wait()
                g_desc(t0 + r, b).start()
                bp = (r - 1) % 3
                pltpu.make_async_copy(
                    xs_hbm.at[idx.at[(r - 1) % 6, 0]], rows.at[bp],
                    gsem[bp]).wait()
                idx_desc(jnp.minimum(t0 + r + 2, T - 1), (r + 2) % 6).start()
                pltpu.async_copy(rows.at[bp], acc.at[idx.at[(r - 1) % 6, 1]],
                                 ssem[bp], add=True)
            return 0

        lax.fori_loop(1, T // 6, body, 0)

        bl = (T - 1) % 3
        g_desc(T - 1, bl).wait()
        s_fire(T - 1, bl)
        for t in (T - 3, T - 2, T - 1):
            s_wait(t, t % 3)
        for u in (T % 6, (T + 1) % 6):
            pltpu.make_async_copy(idx3_hbm.at[wid, T - 1], idx.at[u],
                                  isem[u]).wait()

        plsc.subcore_barrier()
        pltpu.sync_copy(acc.at[pl.ds(s * rpt, rpt)],
                        out_hbm.at[c, pl.ds(s * rpt, rpt)])

    return k


def _sc_deg(NP, e_per_w, n_chunks):
    mesh = plsc.VectorSubcoreMesh(core_axis_name="c", subcore_axis_name="s")
    rpt = NP // NS
    n_super = n_chunks // NB
    ZR = 16

    @functools.partial(
        pl.kernel,
        mesh=mesh,
        out_type=jax.ShapeDtypeStruct((NC, NP, DC), jnp.float32),
        scratch_types=[
            pltpu.VMEM_SHARED((NP, DC), jnp.float32),
            pltpu.VMEM((n_chunks, CHUNK), jnp.int32),
            pltpu.VMEM((CHUNK, DC), jnp.float32),
            pltpu.VMEM((ZR, DC), jnp.float32),
        ] + [pltpu.SemaphoreType.DMA] * NB,
    )
    def k(dst3_hbm, out_hbm, acc, didx, ones, zbuf, *ssem):
        c = lax.axis_index("c")
        s = lax.axis_index("s")
        wid = c * NS + s

        pltpu.sync_copy(dst3_hbm.at[wid], didx)

        def fill_ones(i, _):
            for j in range(DC // L):
                ones[i, pl.ds(j * L, L)] = jnp.ones((L,), jnp.float32)
            return 0

        def fill_zero(i, _):
            for j in range(DC // L):
                zbuf[i, pl.ds(j * L, L)] = jnp.zeros((L,), jnp.float32)
            return 0

        lax.fori_loop(0, CHUNK, fill_ones, 0)
        lax.fori_loop(0, ZR, fill_zero, 0)

        def zero_blk(j, _):
            pltpu.sync_copy(zbuf, acc.at[pl.ds(s * rpt + j * ZR, ZR)])
            return 0

        lax.fori_loop(0, rpt // ZR, zero_blk, 0)
        if rpt % ZR:
            pltpu.sync_copy(zbuf.at[pl.ds(0, rpt % ZR)],
                            acc.at[pl.ds(s * rpt + (rpt // ZR) * ZR,
                                         rpt % ZR)])
        plsc.subcore_barrier()

        def scatter(t, b):
            pltpu.async_copy(ones, acc.at[didx.at[t]], ssem[b], add=True)

        def wait_s(t, b):
            pltpu.make_async_copy(ones, acc.at[didx.at[t]], ssem[b]).wait()

        for b in range(NB):
            scatter(b, b)

        def body(g, _):
            for b in range(NB):
                t = g * NB + b
                wait_s(t, b)
                scatter(t + NB, b)
            return 0

        lax.fori_loop(0, n_super - 1, body, 0)
        for b in range(NB):
            t = (n_super - 1) * NB + b
            wait_s(t, b)

        plsc.subcore_barrier()
        pltpu.sync_copy(acc.at[pl.ds(s * rpt, rpt)],
                        out_hbm.at[c, pl.ds(s * rpt, rpt)])

    return k


def _tc_scale(x_blk, dp_blk, xs_out, dinv_out):
    deg = 1.0 + dp_blk[0, :, :1] + dp_blk[1, :, :1]
    dinv = lax.rsqrt(deg)
    dinv_out[...] = jnp.broadcast_to(dinv, xs_out.shape)
    xs_out[...] = x_blk[...] * dinv


def _tc_mid(p_blk, xs, dinv, W1, b1, W2, ts_out):
    ax = dinv[...] * (p_blk[0] + p_blk[1] + xs[...])
    h1 = jnp.maximum(
        jnp.dot(ax, W1[...], preferred_element_type=jnp.float32) + b1[...], 0.0)
    t = jnp.dot(h1, W2[...], preferred_element_type=jnp.float32)
    ts_out[...] = t * dinv[...]


def _tc_final(q_blk, ts, dinv, b2, out):
    out[...] = dinv[...] * (q_blk[0] + q_blk[1] + ts[...]) + b2[...]


def kernel(x, edge_index, W1, b1, W2, b2):
    N, d_in = x.shape
    E = edge_index.shape[1]
    d_hid = W1.shape[1]
    d_out = W2.shape[1]

    NP = ((N + 1 + NS * 8 - 1) // (NS * 8)) * (NS * 8)
    q6 = NW * CHUNK * 6
    Epad = ((E + q6 - 1) // q6) * q6
    e_per_w = Epad // NW
    n_chunks = e_per_w // CHUNK
    assert n_chunks % 6 == 0 and n_chunks % NB == 0

    pad_idx = (N + jnp.arange(Epad - E, dtype=jnp.int32) % (NP - N)).astype(
        jnp.int32)
    src = jnp.concatenate([edge_index[0], pad_idx])
    dst = jnp.concatenate([edge_index[1], pad_idx])
    src3 = src.reshape(NW, n_chunks, CHUNK)
    dst3 = dst.reshape(NW, n_chunks, CHUNK)
    idx3 = jnp.stack([src3, dst3], axis=2)
    x_pad = jnp.pad(x, ((0, NP - N), (0, 0)))

    degp = _sc_deg(NP, e_per_w, n_chunks)(dst3)

    R = 512
    grid = (pl.cdiv(NP, R),)
    blk = lambda d: pl.BlockSpec((R, d), lambda i: (i, 0))
    blk2 = lambda d: pl.BlockSpec((2, R, d), lambda i: (0, i, 0))
    full = lambda shape: pl.BlockSpec(shape, lambda i: tuple(0 for _ in shape))

    xs, dinv = pl.pallas_call(
        _tc_scale,
        grid=grid,
        in_specs=[blk(d_in), blk2(DC)],
        out_specs=[blk(d_in), blk(d_in)],
        out_shape=[jax.ShapeDtypeStruct((NP, d_in), jnp.float32),
                   jax.ShapeDtypeStruct((NP, d_in), jnp.float32)],
    )(x_pad, degp)

    p = _sc_agg(NP, d_in, e_per_w, n_chunks)(xs, idx3)

    ts = pl.pallas_call(
        _tc_mid,
        grid=grid,
        in_specs=[blk2(d_in), blk(d_in), blk(d_in),
                  full((d_in, d_hid)), full((1, d_hid)), full((d_hid, d_out))],
        out_specs=blk(d_out),
        out_shape=jax.ShapeDtypeStruct((NP, d_out), jnp.float32),
    )(p, xs, dinv, W1, b1.reshape(1, d_hid), W2)

    q = _sc_agg(NP, d_out, e_per_w, n_chunks)(ts, idx3)

    out = pl.pallas_call(
        _tc_final,
        grid=grid,
        in_specs=[blk2(d_out), blk(d_out), blk(d_out), full((1, d_out))],
        out_specs=blk(d_out),
        out_shape=jax.ShapeDtypeStruct((NP, d_out), jnp.float32),
    )(q, ts, dinv, b2.reshape(1, d_out))

    return out[:N]


# speedup vs baseline: 32.7721x; 1.0445x over previous
import functools

import jax
import jax.numpy as jnp
from jax import lax
from jax.experimental import pallas as pl
from jax.experimental.pallas import tpu as pltpu
from jax.experimental.pallas import tpu_sc as plsc

NC = 2
NS = 16
L = 16
NW = NC * NS
CHUNK = 120
DC = 128
NB = 4


def _sc_agg(NP, D, e_per_w, n_chunks):
    mesh = plsc.VectorSubcoreMesh(core_axis_name="c", subcore_axis_name="s")
    rpt = NP // NS
    T = n_chunks
    ZC = rpt // CHUNK
    ZT = rpt - ZC * CHUNK

    @functools.partial(
        pl.kernel,
        mesh=mesh,
        out_type=jax.ShapeDtypeStruct((NC, NP, D), jnp.float32),
        scratch_types=[
            pltpu.VMEM_SHARED((NP, D), jnp.float32),
            pltpu.VMEM((6, 2, CHUNK), jnp.int32),
            pltpu.VMEM((3, CHUNK, D), jnp.float32),
        ] + [pltpu.SemaphoreType.DMA] * 12,
    )
    def k(xs_hbm, idx3_hbm, out_hbm, acc, idx, rows, *sems):
        isem, gsem, ssem = sems[0:6], sems[6:9], sems[9:12]
        c = lax.axis_index("c")
        s = lax.axis_index("s")
        wid = c * NS + s

        def fill_zero(i, _):
            for j in range(D // L):
                rows[0, i, pl.ds(j * L, L)] = jnp.zeros((L,), jnp.float32)
            return 0

        lax.fori_loop(0, CHUNK, fill_zero, 0)
        for j in range(ZC):
            pltpu.sync_copy(rows.at[0],
                            acc.at[pl.ds(s * rpt + j * CHUNK, CHUNK)])
        if ZT:
            pltpu.sync_copy(rows.at[0, pl.ds(0, ZT)],
                            acc.at[pl.ds(s * rpt + ZC * CHUNK, ZT)])
        plsc.subcore_barrier()

        def idx_desc(t, i6):
            return pltpu.make_async_copy(idx3_hbm.at[wid, t], idx.at[i6],
                                         isem[i6])

        def g_desc(t, b):
            return pltpu.make_async_copy(xs_hbm.at[idx.at[t % 6, 0]],
                                         rows.at[b], gsem[b])

        def s_fire(t, b):
            pltpu.async_copy(rows.at[b], acc.at[idx.at[t % 6, 1]], ssem[b],
                             add=True)

        def s_wait(t, b):
            pltpu.make_async_copy(rows.at[b], acc.at[idx.at[t % 6, 1]],
                                  ssem[b]).wait()

        def visit(t):
            b = t % 3
            s_wait(t - 3, b)
            idx_desc(t, t % 6).wait()
            g_desc(t, b).start()
            g_desc(t - 1, (t - 1) % 3).wait()
            idx_desc(min(t + 2, T - 1), (t + 2) % 6).start()
            s_fire(t - 1, (t - 1) % 3)

        for t in range(3):
            idx_desc(t, t).start()
        idx_desc(0, 0).wait()
        g_desc(0, 0).start()
        for t in (1, 2):
            idx_desc(t, t).wait()
            g_desc(t, t).start()
            idx_desc(t + 2, (t + 2) % 6).start()
            g_desc(t - 1, t - 1).wait()
            s_fire(t - 1, t - 1)
        for t in (3, 4, 5):
            visit(t)

        def body(g, _):
            t0 = 6 * g
            for r in range(6):
                b = r % 3
                i6p = (t0 + r - 3) % 6
                pltpu.make_async_copy(rows.at[b], acc.at[idx.at[i6p, 1]],
                                      ssem[b]).wait()
                idx_desc(t0 + r, r).wait()
                g_desc(t0 + r, b).start()
                bp = (r - 1) % 3
                pltpu.make_async_copy(
                    xs_hbm.at[idx.at[(r - 1) % 6, 0]], rows.at[bp],
                    gsem[bp]).wait()
                idx_desc(jnp.minimum(t0 + r + 2, T - 1), (r + 2) % 6).start()
                pltpu.async_copy(rows.at[bp], acc.at[idx.at[(r - 1) % 6, 1]],
                                 ssem[bp], add=True)
            return 0

        lax.fori_loop(1, T // 6, body, 0)

        bl = (T - 1) % 3
        g_desc(T - 1, bl).wait()
        s_fire(T - 1, bl)
        for t in (T - 3, T - 2, T - 1):
            s_wait(t, t % 3)
        for u in (T % 6, (T + 1) % 6):
            pltpu.make_async_copy(idx3_hbm.at[wid, T - 1], idx.at[u],
                                  isem[u]).wait()

        plsc.subcore_barrier()
        pltpu.sync_copy(acc.at[pl.ds(s * rpt, rpt)],
                        out_hbm.at[c, pl.ds(s * rpt, rpt)])

    return k


def _sc_deg(NP, e_per_w, n_chunks):
    mesh = plsc.VectorSubcoreMesh(core_axis_name="c", subcore_axis_name="s")
    rpt = NP // NS
    n_super = n_chunks // NB
    ZR = 16

    @functools.partial(
        pl.kernel,
        mesh=mesh,
        out_type=jax.ShapeDtypeStruct((NC, NP, DC), jnp.float32),
        scratch_types=[
            pltpu.VMEM_SHARED((NP, DC), jnp.float32),
            pltpu.VMEM((n_chunks, CHUNK), jnp.int32),
            pltpu.VMEM((CHUNK, DC), jnp.float32),
            pltpu.VMEM((ZR, DC), jnp.float32),
        ] + [pltpu.SemaphoreType.DMA] * NB,
    )
    def k(dst3_hbm, out_hbm, acc, didx, ones, zbuf, *ssem):
        c = lax.axis_index("c")
        s = lax.axis_index("s")
        wid = c * NS + s

        pltpu.sync_copy(dst3_hbm.at[wid], didx)

        def fill_ones(i, _):
            for j in range(DC // L):
                ones[i, pl.ds(j * L, L)] = jnp.ones((L,), jnp.float32)
            return 0

        def fill_zero(i, _):
            for j in range(DC // L):
                zbuf[i, pl.ds(j * L, L)] = jnp.zeros((L,), jnp.float32)
            return 0

        lax.fori_loop(0, CHUNK, fill_ones, 0)
        lax.fori_loop(0, ZR, fill_zero, 0)

        def zero_blk(j, _):
            pltpu.sync_copy(zbuf, acc.at[pl.ds(s * rpt + j * ZR, ZR)])
            return 0

        lax.fori_loop(0, rpt // ZR, zero_blk, 0)
        if rpt % ZR:
            pltpu.sync_copy(zbuf.at[pl.ds(0, rpt % ZR)],
                            acc.at[pl.ds(s * rpt + (rpt // ZR) * ZR,
                                         rpt % ZR)])
        plsc.subcore_barrier()

        def scatter(t, b):
            pltpu.async_copy(ones, acc.at[didx.at[t]], ssem[b], add=True)

        def wait_s(t, b):
            pltpu.make_async_copy(ones, acc.at[didx.at[t]], ssem[b]).wait()

        for b in range(NB):
            scatter(b, b)

        def body(g, _):
            for b in range(NB):
                t = g * NB + b
                wait_s(t, b)
                scatter(t + NB, b)
            return 0

        lax.fori_loop(0, n_super - 1, body, 0)
        for b in range(NB):
            t = (n_super - 1) * NB + b
            wait_s(t, b)

        plsc.subcore_barrier()
        pltpu.sync_copy(acc.at[pl.ds(s * rpt, rpt)],
                        out_hbm.at[c, pl.ds(s * rpt, rpt)])

    return k


def _tc_scale(x_blk, dp_blk, xs_out, dinv_out):
    deg = 1.0 + dp_blk[0, :, :1] + dp_blk[1, :, :1]
    dinv = lax.rsqrt(deg)
    dinv_out[...] = jnp.broadcast_to(dinv, xs_out.shape)
    xs_out[...] = x_blk[...] * dinv


def _tc_mid(p_blk, xs, dinv, W1, b1, W2, ts_out):
    ax = dinv[...] * (p_blk[0] + p_blk[1] + xs[...])
    h1 = jnp.maximum(
        jnp.dot(ax, W1[...], preferred_element_type=jnp.float32) + b1[...], 0.0)
    t = jnp.dot(h1, W2[...], preferred_element_type=jnp.float32)
    ts_out[...] = t * dinv[...]


def _tc_final(q_blk, ts, dinv, b2, out):
    out[...] = dinv[...] * (q_blk[0] + q_blk[1] + ts[...]) + b2[...]


def kernel(x, edge_index, W1, b1, W2, b2):
    N, d_in = x.shape
    E = edge_index.shape[1]
    d_hid = W1.shape[1]
    d_out = W2.shape[1]

    NP = ((N + 1 + NS * 8 - 1) // (NS * 8)) * (NS * 8)
    q6 = NW * CHUNK * 6
    Epad = ((E + q6 - 1) // q6) * q6
    e_per_w = Epad // NW
    n_chunks = e_per_w // CHUNK
    assert n_chunks % 6 == 0 and n_chunks % NB == 0

    pad_idx = (N + jnp.arange(Epad - E, dtype=jnp.int32) % (NP - N)).astype(
        jnp.int32)
    src = jnp.concatenate([edge_index[0], pad_idx])
    dst = jnp.concatenate([edge_index[1], pad_idx])
    src3 = src.reshape(NW, n_chunks, CHUNK)
    dst3 = dst.reshape(NW, n_chunks, CHUNK)
    idx3 = jnp.stack([src3, dst3], axis=2)
    x_pad = jnp.pad(x, ((0, NP - N), (0, 0)))

    degp = _sc_deg(NP, e_per_w, n_chunks)(dst3)

    R = 1024
    grid = (pl.cdiv(NP, R),)
    blk = lambda d: pl.BlockSpec((R, d), lambda i: (i, 0))
    blk2 = lambda d: pl.BlockSpec((2, R, d), lambda i: (0, i, 0))
    full = lambda shape: pl.BlockSpec(shape, lambda i: tuple(0 for _ in shape))

    xs, dinv = pl.pallas_call(
        _tc_scale,
        grid=grid,
        in_specs=[blk(d_in), blk2(DC)],
        out_specs=[blk(d_in), blk(d_in)],
        out_shape=[jax.ShapeDtypeStruct((NP, d_in), jnp.float32),
                   jax.ShapeDtypeStruct((NP, d_in), jnp.float32)],
    )(x_pad, degp)

    p = _sc_agg(NP, d_in, e_per_w, n_chunks)(xs, idx3)

    ts = pl.pallas_call(
        _tc_mid,
        grid=grid,
        in_specs=[blk2(d_in), blk(d_in), blk(d_in),
                  full((d_in, d_hid)), full((1, d_hid)), full((d_hid, d_out))],
        out_specs=blk(d_out),
        out_shape=jax.ShapeDtypeStruct((NP, d_out), jnp.float32),
    )(p, xs, dinv, W1, b1.reshape(1, d_hid), W2)

    q = _sc_agg(NP, d_out, e_per_w, n_chunks)(ts, idx3)

    out = pl.pallas_call(
        _tc_final,
        grid=grid,
        in_specs=[blk2(d_out), blk(d_out), blk(d_out), full((1, d_out))],
        out_specs=blk(d_out),
        out_shape=jax.ShapeDtypeStruct((NP, d_out), jnp.float32),
    )(q, ts, dinv, b2.reshape(1, d_out))

    return out[:N]


# speedup vs baseline: 33.3688x; 1.0182x over previous
import functools

import jax
import jax.numpy as jnp
from jax import lax
from jax.experimental import pallas as pl
from jax.experimental.pallas import tpu as pltpu
from jax.experimental.pallas import tpu_sc as plsc

NC = 2
NS = 16
L = 16
NW = NC * NS
CHUNK = 120
DC = 128
NB = 4


def _sc_agg(NP, D, e_per_w, n_chunks):
    mesh = plsc.VectorSubcoreMesh(core_axis_name="c", subcore_axis_name="s")
    rpt = NP // NS
    T = n_chunks
    ZC = rpt // CHUNK
    ZT = rpt - ZC * CHUNK

    @functools.partial(
        pl.kernel,
        mesh=mesh,
        out_type=jax.ShapeDtypeStruct((NC, NP, D), jnp.float32),
        scratch_types=[
            pltpu.VMEM_SHARED((NP, D), jnp.float32),
            pltpu.VMEM((6, 2, CHUNK), jnp.int32),
            pltpu.VMEM((3, CHUNK, D), jnp.float32),
        ] + [pltpu.SemaphoreType.DMA] * 12,
    )
    def k(xs_hbm, idx3_hbm, out_hbm, acc, idx, rows, *sems):
        isem, gsem, ssem = sems[0:6], sems[6:9], sems[9:12]
        c = lax.axis_index("c")
        s = lax.axis_index("s")
        wid = c * NS + s

        def fill_zero(i, _):
            for j in range(D // L):
                rows[0, i, pl.ds(j * L, L)] = jnp.zeros((L,), jnp.float32)
            return 0

        lax.fori_loop(0, CHUNK, fill_zero, 0)
        for j in range(ZC):
            pltpu.sync_copy(rows.at[0],
                            acc.at[pl.ds(s * rpt + j * CHUNK, CHUNK)])
        if ZT:
            pltpu.sync_copy(rows.at[0, pl.ds(0, ZT)],
                            acc.at[pl.ds(s * rpt + ZC * CHUNK, ZT)])
        plsc.subcore_barrier()

        def idx_desc(t, i6):
            return pltpu.make_async_copy(idx3_hbm.at[wid, t], idx.at[i6],
                                         isem[i6])

        def g_desc(t, b):
            return pltpu.make_async_copy(xs_hbm.at[idx.at[t % 6, 0]],
                                         rows.at[b], gsem[b])

        def s_fire(t, b):
            pltpu.async_copy(rows.at[b], acc.at[idx.at[t % 6, 1]], ssem[b],
                             add=True)

        def s_wait(t, b):
            pltpu.make_async_copy(rows.at[b], acc.at[idx.at[t % 6, 1]],
                                  ssem[b]).wait()

        def visit(t):
            b = t % 3
            s_wait(t - 3, b)
            idx_desc(t, t % 6).wait()
            g_desc(t, b).start()
            g_desc(t - 1, (t - 1) % 3).wait()
            idx_desc(min(t + 2, T - 1), (t + 2) % 6).start()
            s_fire(t - 1, (t - 1) % 3)

        for t in range(3):
            idx_desc(t, t).start()
        idx_desc(0, 0).wait()
        g_desc(0, 0).start()
        for t in (1, 2):
            idx_desc(t, t).wait()
            g_desc(t, t).start()
            idx_desc(t + 2, (t + 2) % 6).start()
            g_desc(t - 1, t - 1).wait()
            s_fire(t - 1, t - 1)
        for t in (3, 4, 5):
            visit(t)

        def body(g, _):
            t0 = 6 * g
            for r in range(6):
                b = r % 3
                i6p = (t0 + r - 3) % 6
                pltpu.make_async_copy(rows.at[b], acc.at[idx.at[i6p, 1]],
                                      ssem[b]).wait()
                idx_desc(t0 + r, r).wait()
                g_desc(t0 + r, b).start()
                bp = (r - 1) % 3
                pltpu.make_async_copy(
                    xs_hbm.at[idx.at[(r - 1) % 6, 0]], rows.at[bp],
                    gsem[bp]).wait()
                idx_desc(jnp.minimum(t0 + r + 2, T - 1), (r + 2) % 6).start()
                pltpu.async_copy(rows.at[bp], acc.at[idx.at[(r - 1) % 6, 1]],
                                 ssem[bp], add=True)
            return 0

        lax.fori_loop(1, T // 6, body, 0)

        bl = (T - 1) % 3
        g_desc(T - 1, bl).wait()
        s_fire(T - 1, bl)
        for t in (T - 3, T - 2, T - 1):
            s_wait(t, t % 3)
        for u in (T % 6, (T + 1) % 6):
            pltpu.make_async_copy(idx3_hbm.at[wid, T - 1], idx.at[u],
                                  isem[u]).wait()

        plsc.subcore_barrier()
        pltpu.sync_copy(acc.at[pl.ds(s * rpt, rpt)],
                        out_hbm.at[c, pl.ds(s * rpt, rpt)])

    return k


def _sc_deg(NP, e_per_w, n_chunks):
    mesh = plsc.VectorSubcoreMesh(core_axis_name="c", subcore_axis_name="s")
    rpt = NP // NS
    n_super = n_chunks // NB
    ZR = 16

    @functools.partial(
        pl.kernel,
        mesh=mesh,
        out_type=jax.ShapeDtypeStruct((NC, NP, DC), jnp.float32),
        scratch_types=[
            pltpu.VMEM_SHARED((NP, DC), jnp.float32),
            pltpu.VMEM((n_chunks, CHUNK), jnp.int32),
            pltpu.VMEM((CHUNK, DC), jnp.float32),
            pltpu.VMEM((ZR, DC), jnp.float32),
        ] + [pltpu.SemaphoreType.DMA] * NB,
    )
    def k(dst3_hbm, out_hbm, acc, didx, ones, zbuf, *ssem):
        c = lax.axis_index("c")
        s = lax.axis_index("s")
        wid = c * NS + s

        pltpu.sync_copy(dst3_hbm.at[wid], didx)

        def fill_ones(i, _):
            for j in range(DC // L):
                ones[i, pl.ds(j * L, L)] = jnp.ones((L,), jnp.float32)
            return 0

        def fill_zero(i, _):
            for j in range(DC // L):
                zbuf[i, pl.ds(j * L, L)] = jnp.zeros((L,), jnp.float32)
            return 0

        lax.fori_loop(0, CHUNK, fill_ones, 0)
        lax.fori_loop(0, ZR, fill_zero, 0)

        def zero_blk(j, _):
            pltpu.sync_copy(zbuf, acc.at[pl.ds(s * rpt + j * ZR, ZR)])
            return 0

        lax.fori_loop(0, rpt // ZR, zero_blk, 0)
        if rpt % ZR:
            pltpu.sync_copy(zbuf.at[pl.ds(0, rpt % ZR)],
                            acc.at[pl.ds(s * rpt + (rpt // ZR) * ZR,
                                         rpt % ZR)])
        plsc.subcore_barrier()

        def scatter(t, b):
            pltpu.async_copy(ones, acc.at[didx.at[t]], ssem[b], add=True)

        def wait_s(t, b):
            pltpu.make_async_copy(ones, acc.at[didx.at[t]], ssem[b]).wait()

        for b in range(NB):
            scatter(b, b)

        def body(g, _):
            for b in range(NB):
                t = g * NB + b
                wait_s(t, b)
                scatter(t + NB, b)
            return 0

        lax.fori_loop(0, n_super - 1, body, 0)
        for b in range(NB):
            t = (n_super - 1) * NB + b
            wait_s(t, b)

        plsc.subcore_barrier()
        pltpu.sync_copy(acc.at[pl.ds(s * rpt, rpt)],
                        out_hbm.at[c, pl.ds(s * rpt, rpt)])

    return k


def _tc_scale(x_blk, dp_blk, xs_out, dinv_out):
    deg = 1.0 + dp_blk[0, :, :1] + dp_blk[1, :, :1]
    dinv = lax.rsqrt(deg)
    dinv_out[...] = jnp.broadcast_to(dinv, xs_out.shape)
    xs_out[...] = x_blk[...] * dinv


def _tc_mid(p_blk, xs, dinv, W1, b1, W2, ts_out):
    ax = dinv[...] * (p_blk[0] + p_blk[1] + xs[...])
    h1 = jnp.maximum(
        jnp.dot(ax, W1[...], preferred_element_type=jnp.float32) + b1[...], 0.0)
    t = jnp.dot(h1, W2[...], preferred_element_type=jnp.float32)
    ts_out[...] = t * dinv[...]


def _tc_final(q_blk, ts, dinv, b2, out):
    out[...] = dinv[...] * (q_blk[0] + q_blk[1] + ts[...]) + b2[...]


def kernel(x, edge_index, W1, b1, W2, b2):
    N, d_in = x.shape
    E = edge_index.shape[1]
    d_hid = W1.shape[1]
    d_out = W2.shape[1]

    NP = ((N + 1 + NS * 8 - 1) // (NS * 8)) * (NS * 8)
    q6 = NW * CHUNK * 6
    Epad = ((E + q6 - 1) // q6) * q6
    e_per_w = Epad // NW
    n_chunks = e_per_w // CHUNK
    assert n_chunks % 6 == 0 and n_chunks % NB == 0

    pad_idx = (N + jnp.arange(Epad - E, dtype=jnp.int32) % (NP - N)).astype(
        jnp.int32)
    src = jnp.concatenate([edge_index[0], pad_idx])
    dst = jnp.concatenate([edge_index[1], pad_idx])
    src3 = src.reshape(NW, n_chunks, CHUNK)
    dst3 = dst.reshape(NW, n_chunks, CHUNK)
    idx3 = jnp.stack([src3, dst3], axis=2)
    x_pad = jnp.pad(x, ((0, NP - N), (0, 0)))

    degp = _sc_deg(NP, e_per_w, n_chunks)(dst3)

    R = 2048
    grid = (pl.cdiv(NP, R),)
    blk = lambda d: pl.BlockSpec((R, d), lambda i: (i, 0))
    blk2 = lambda d: pl.BlockSpec((2, R, d), lambda i: (0, i, 0))
    full = lambda shape: pl.BlockSpec(shape, lambda i: tuple(0 for _ in shape))

    xs, dinv = pl.pallas_call(
        _tc_scale,
        grid=grid,
        in_specs=[blk(d_in), blk2(DC)],
        out_specs=[blk(d_in), blk(d_in)],
        out_shape=[jax.ShapeDtypeStruct((NP, d_in), jnp.float32),
                   jax.ShapeDtypeStruct((NP, d_in), jnp.float32)],
    )(x_pad, degp)

    p = _sc_agg(NP, d_in, e_per_w, n_chunks)(xs, idx3)

    ts = pl.pallas_call(
        _tc_mid,
        grid=grid,
        in_specs=[blk2(d_in), blk(d_in), blk(d_in),
                  full((d_in, d_hid)), full((1, d_hid)), full((d_hid, d_out))],
        out_specs=blk(d_out),
        out_shape=jax.ShapeDtypeStruct((NP, d_out), jnp.float32),
    )(p, xs, dinv, W1, b1.reshape(1, d_hid), W2)

    q = _sc_agg(NP, d_out, e_per_w, n_chunks)(ts, idx3)

    out = pl.pallas_call(
        _tc_final,
        grid=grid,
        in_specs=[blk2(d_out), blk(d_out), blk(d_out), full((1, d_out))],
        out_specs=blk(d_out),
        out_shape=jax.ShapeDtypeStruct((NP, d_out), jnp.float32),
    )(q, ts, dinv, b2.reshape(1, d_out))

    return out[:N]


# speedup vs baseline: 33.7433x; 1.0112x over previous
import functools

import jax
import jax.numpy as jnp
from jax import lax
from jax.experimental import pallas as pl
from jax.experimental.pallas import tpu as pltpu
from jax.experimental.pallas import tpu_sc as plsc

NC = 2
NS = 16
L = 16
NW = NC * NS
CHUNK = 120
DC = 128
NB = 4


def _sc_agg(NP, D, e_per_w, n_chunks):
    mesh = plsc.VectorSubcoreMesh(core_axis_name="c", subcore_axis_name="s")
    rpt = NP // NS
    T = n_chunks
    ZC = rpt // CHUNK
    ZT = rpt - ZC * CHUNK

    @functools.partial(
        pl.kernel,
        mesh=mesh,
        out_type=jax.ShapeDtypeStruct((NC, NP, D), jnp.float32),
        scratch_types=[
            pltpu.VMEM_SHARED((NP, D), jnp.float32),
            pltpu.VMEM((6, 2, CHUNK), jnp.int32),
            pltpu.VMEM((3, CHUNK, D), jnp.float32),
        ] + [pltpu.SemaphoreType.DMA] * 12,
    )
    def k(xs_hbm, idx3_hbm, out_hbm, acc, idx, rows, *sems):
        isem, gsem, ssem = sems[0:6], sems[6:9], sems[9:12]
        c = lax.axis_index("c")
        s = lax.axis_index("s")
        wid = c * NS + s

        def fill_zero(i, _):
            for j in range(D // L):
                rows[0, i, pl.ds(j * L, L)] = jnp.zeros((L,), jnp.float32)
            return 0

        lax.fori_loop(0, CHUNK, fill_zero, 0)
        for j in range(ZC):
            pltpu.sync_copy(rows.at[0],
                            acc.at[pl.ds(s * rpt + j * CHUNK, CHUNK)])
        if ZT:
            pltpu.sync_copy(rows.at[0, pl.ds(0, ZT)],
                            acc.at[pl.ds(s * rpt + ZC * CHUNK, ZT)])
        plsc.subcore_barrier()

        def idx_desc(t, i6):
            return pltpu.make_async_copy(idx3_hbm.at[wid, t], idx.at[i6],
                                         isem[i6])

        def g_desc(t, b):
            return pltpu.make_async_copy(xs_hbm.at[idx.at[t % 6, 0]],
                                         rows.at[b], gsem[b])

        def s_fire(t, b):
            pltpu.async_copy(rows.at[b], acc.at[idx.at[t % 6, 1]], ssem[b],
                             add=True)

        def s_wait(t, b):
            pltpu.make_async_copy(rows.at[b], acc.at[idx.at[t % 6, 1]],
                                  ssem[b]).wait()

        def visit(t):
            b = t % 3
            s_wait(t - 3, b)
            idx_desc(t, t % 6).wait()
            g_desc(t, b).start()
            g_desc(t - 1, (t - 1) % 3).wait()
            idx_desc(min(t + 2, T - 1), (t + 2) % 6).start()
            s_fire(t - 1, (t - 1) % 3)

        for t in range(3):
            idx_desc(t, t).start()
        idx_desc(0, 0).wait()
        g_desc(0, 0).start()
        for t in (1, 2):
            idx_desc(t, t).wait()
            g_desc(t, t).start()
            idx_desc(t + 2, (t + 2) % 6).start()
            g_desc(t - 1, t - 1).wait()
            s_fire(t - 1, t - 1)
        for t in (3, 4, 5):
            visit(t)

        def body(g, _):
            t0 = 6 * g
            for r in range(6):
                b = r % 3
                i6p = (t0 + r - 3) % 6
                pltpu.make_async_copy(rows.at[b], acc.at[idx.at[i6p, 1]],
                                      ssem[b]).wait()
                idx_desc(t0 + r, r).wait()
                g_desc(t0 + r, b).start()
                bp = (r - 1) % 3
                pltpu.make_async_copy(
                    xs_hbm.at[idx.at[(r - 1) % 6, 0]], rows.at[bp],
                    gsem[bp]).wait()
                idx_desc(jnp.minimum(t0 + r + 2, T - 1), (r + 2) % 6).start()
                pltpu.async_copy(rows.at[bp], acc.at[idx.at[(r - 1) % 6, 1]],
                                 ssem[bp], add=True)
            return 0

        lax.fori_loop(1, T // 6, body, 0)

        bl = (T - 1) % 3
        g_desc(T - 1, bl).wait()
        s_fire(T - 1, bl)
        for t in (T - 3, T - 2, T - 1):
            s_wait(t, t % 3)
        for u in (T % 6, (T + 1) % 6):
            pltpu.make_async_copy(idx3_hbm.at[wid, T - 1], idx.at[u],
                                  isem[u]).wait()

        plsc.subcore_barrier()
        pltpu.sync_copy(acc.at[pl.ds(s * rpt, rpt)],
                        out_hbm.at[c, pl.ds(s * rpt, rpt)])

    return k


def _sc_deg(NP, e_per_w, n_chunks):
    mesh = plsc.VectorSubcoreMesh(core_axis_name="c", subcore_axis_name="s")
    rpt = NP // NS
    n_super = n_chunks // NB
    ZR = 16

    @functools.partial(
        pl.kernel,
        mesh=mesh,
        out_type=jax.ShapeDtypeStruct((NC, NP, DC), jnp.float32),
        scratch_types=[
            pltpu.VMEM_SHARED((NP, DC), jnp.float32),
            pltpu.VMEM((n_chunks, CHUNK), jnp.int32),
            pltpu.VMEM((CHUNK, DC), jnp.float32),
            pltpu.VMEM((ZR, DC), jnp.float32),
        ] + [pltpu.SemaphoreType.DMA] * NB,
    )
    def k(dst3_hbm, out_hbm, acc, didx, ones, zbuf, *ssem):
        c = lax.axis_index("c")
        s = lax.axis_index("s")
        wid = c * NS + s

        pltpu.sync_copy(dst3_hbm.at[wid], didx)

        def fill_ones(i, _):
            for j in range(DC // L):
                ones[i, pl.ds(j * L, L)] = jnp.ones((L,), jnp.float32)
            return 0

        def fill_zero(i, _):
            for j in range(DC // L):
                zbuf[i, pl.ds(j * L, L)] = jnp.zeros((L,), jnp.float32)
            return 0

        lax.fori_loop(0, CHUNK, fill_ones, 0)
        lax.fori_loop(0, ZR, fill_zero, 0)

        def zero_blk(j, _):
            pltpu.sync_copy(zbuf, acc.at[pl.ds(s * rpt + j * ZR, ZR)])
            return 0

        lax.fori_loop(0, rpt // ZR, zero_blk, 0)
        if rpt % ZR:
            pltpu.sync_copy(zbuf.at[pl.ds(0, rpt % ZR)],
                            acc.at[pl.ds(s * rpt + (rpt // ZR) * ZR,
                                         rpt % ZR)])
        plsc.subcore_barrier()

        def scatter(t, b):
            pltpu.async_copy(ones, acc.at[didx.at[t]], ssem[b], add=True)

        def wait_s(t, b):
            pltpu.make_async_copy(ones, acc.at[didx.at[t]], ssem[b]).wait()

        for b in range(NB):
            scatter(b, b)

        def body(g, _):
            for b in range(NB):
                t = g * NB + b
                wait_s(t, b)
                scatter(t + NB, b)
            return 0

        lax.fori_loop(0, n_super - 1, body, 0)
        for b in range(NB):
            t = (n_super - 1) * NB + b
            wait_s(t, b)

        plsc.subcore_barrier()
        pltpu.sync_copy(acc.at[pl.ds(s * rpt, rpt)],
                        out_hbm.at[c, pl.ds(s * rpt, rpt)])

    return k


def _tc_scale(x_blk, dp_blk, xs_out, dinv_out):
    deg = 1.0 + dp_blk[0, :, :1] + dp_blk[1, :, :1]
    dinv = lax.rsqrt(deg)
    dinv_out[...] = jnp.broadcast_to(dinv, xs_out.shape)
    xs_out[...] = x_blk[...] * dinv


def _tc_mid(p_blk, xs, dinv, W1, b1, W2, ts_out):
    ax = dinv[...] * (p_blk[0] + p_blk[1] + xs[...])
    h1 = jnp.maximum(
        jnp.dot(ax, W1[...], preferred_element_type=jnp.float32) + b1[...], 0.0)
    t = jnp.dot(h1, W2[...], preferred_element_type=jnp.float32)
    ts_out[...] = t * dinv[...]


def _tc_final(q_blk, ts, dinv, b2, out):
    out[...] = dinv[...] * (q_blk[0] + q_blk[1] + ts[...]) + b2[...]


def kernel(x, edge_index, W1, b1, W2, b2):
    N, d_in = x.shape
    E = edge_index.shape[1]
    d_hid = W1.shape[1]
    d_out = W2.shape[1]

    NP = ((N + 1 + NS * 8 - 1) // (NS * 8)) * (NS * 8)
    q6 = NW * CHUNK * 6
    Epad = ((E + q6 - 1) // q6) * q6
    e_per_w = Epad // NW
    n_chunks = e_per_w // CHUNK
    assert n_chunks % 6 == 0 and n_chunks % NB == 0

    pad_idx = (N + jnp.arange(Epad - E, dtype=jnp.int32) % (NP - N)).astype(
        jnp.int32)
    src = jnp.concatenate([edge_index[0], pad_idx])
    dst = jnp.concatenate([edge_index[1], pad_idx])
    src3 = src.reshape(NW, n_chunks, CHUNK)
    dst3 = dst.reshape(NW, n_chunks, CHUNK)
    idx3 = jnp.stack([src3, dst3], axis=2)
    x_pad = jnp.pad(x, ((0, NP - N), (0, 0)))

    degp = _sc_deg(NP, e_per_w, n_chunks)(dst3)

    R = 4096
    grid = (pl.cdiv(NP, R),)
    blk = lambda d: pl.BlockSpec((R, d), lambda i: (i, 0))
    blk2 = lambda d: pl.BlockSpec((2, R, d), lambda i: (0, i, 0))
    full = lambda shape: pl.BlockSpec(shape, lambda i: tuple(0 for _ in shape))

    xs, dinv = pl.pallas_call(
        _tc_scale,
        grid=grid,
        in_specs=[blk(d_in), blk2(DC)],
        out_specs=[blk(d_in), blk(d_in)],
        out_shape=[jax.ShapeDtypeStruct((NP, d_in), jnp.float32),
                   jax.ShapeDtypeStruct((NP, d_in), jnp.float32)],
    )(x_pad, degp)

    p = _sc_agg(NP, d_in, e_per_w, n_chunks)(xs, idx3)

    ts = pl.pallas_call(
        _tc_mid,
        grid=grid,
        in_specs=[blk2(d_in), blk(d_in), blk(d_in),
                  full((d_in, d_hid)), full((1, d_hid)), full((d_hid, d_out))],
        out_specs=blk(d_out),
        out_shape=jax.ShapeDtypeStruct((NP, d_out), jnp.float32),
    )(p, xs, dinv, W1, b1.reshape(1, d_hid), W2)

    q = _sc_agg(NP, d_out, e_per_w, n_chunks)(ts, idx3)

    out = pl.pallas_call(
        _tc_final,
        grid=grid,
        in_specs=[blk2(d_out), blk(d_out), blk(d_out), full((1, d_out))],
        out_specs=blk(d_out),
        out_shape=jax.ShapeDtypeStruct((NP, d_out), jnp.float32),
    )(q, ts, dinv, b2.reshape(1, d_out))

    return out[:N]
